# baseline (speedup 1.0000x reference)
import functools

import jax
import jax.numpy as jnp
from jax import lax
from jax.experimental import pallas as pl
from jax.experimental.pallas import tpu as pltpu
from jax.experimental.pallas import tpu_sc as plsc

DIM = 128
NUM_CORES = 2
NUM_SUBCORES = 16
NUM_WORKERS = NUM_CORES * NUM_SUBCORES
CHUNK = 128
CPG = 2
NG = 3
GROUP = CPG * CHUNK


def _make_gather(total, vocab):
    assert total % (NUM_WORKERS * GROUP) == 0
    per_worker = total // NUM_WORKERS
    n_chunks = per_worker // CHUNK
    n_groups = n_chunks // CPG
    n_steady = (n_groups // NG) - 1
    assert n_steady >= 1

    mesh = plsc.VectorSubcoreMesh(
        core_axis_name="c",
        subcore_axis_name="s",
        num_cores=NUM_CORES,
        num_subcores=NUM_SUBCORES,
    )

    @functools.partial(
        pl.kernel,
        out_type=jax.ShapeDtypeStruct((total, DIM), jnp.float32),
        mesh=mesh,
        scratch_types=[
            pltpu.VMEM((n_chunks, CHUNK), jnp.int32),
            pltpu.VMEM((NG * GROUP, DIM), jnp.float32),
            [pltpu.SemaphoreType.DMA for _ in range(NG)],
            [pltpu.SemaphoreType.DMA for _ in range(NG)],
        ],
    )
    def gather(idx_hbm, table_hbm, out_hbm, idx_v, rows, gsem, osem):
        wid = lax.axis_index("s") * NUM_CORES + lax.axis_index("c")
        base = wid * per_worker

        pltpu.sync_copy(idx_hbm.at[pl.ds(wid * n_chunks, n_chunks)], idx_v)

        def gathers(p, b):
            return [
                pltpu.make_async_copy(
                    table_hbm.at[pl.ds((p * CPG + j) % 512 * CHUNK, CHUNK)],
                    rows.at[pl.ds(b * GROUP + j * CHUNK, CHUNK)],
                    gsem[b],
                )
                for j in range(CPG)
            ]

        def out_copy(p, b):
            return pltpu.make_async_copy(
                rows.at[pl.ds(b * GROUP, GROUP)],
                out_hbm.at[pl.ds(base + p * GROUP, GROUP)],
                osem[b],
            )

        def step(p, b, first_round):
            if not first_round:
                out_copy(p - NG, b).wait()
            for c in gathers(p, b):
                c.start()
            if not (first_round and b == 0):
                u, bu = p - 1, (b - 1) % NG
                for c in gathers(u, bu):
                    c.wait()
                out_copy(u, bu).start()

        for b in range(NG):
            step(b, b, first_round=True)

        def outer(g):
            for b in range(NG):
                step(g * NG + b, b, first_round=False)

        pl.loop(1, n_steady + 1)(outer)

        for p in range((n_steady + 1) * NG, n_groups):
            step(p, p % NG, first_round=False)
        u, bu = n_groups - 1, (n_groups - 1) % NG
        for c in gathers(u, bu):
            c.wait()
        out_copy(u, bu).start()
        for k in range(NG):
            p = n_groups - NG + k
            out_copy(p, p % NG).wait()

    return gather


def kernel(inputs, embeddings):
    batch, hist = inputs.shape
    idx = inputs.reshape(-1).astype(jnp.int32)
    total = idx.shape[0]
    idx2d = idx.reshape(total // CHUNK, CHUNK)
    out = _make_gather(total, embeddings.shape[0])(idx2d, embeddings)
    return out.reshape(batch, hist, DIM)


# speedup vs baseline: 1.2379x; 1.2379x over previous
import functools

import jax
import jax.numpy as jnp
from jax import lax
from jax.experimental import pallas as pl
from jax.experimental.pallas import tpu as pltpu
from jax.experimental.pallas import tpu_sc as plsc

DIM = 128
NUM_CORES = 2
NUM_SUBCORES = 16
NUM_WORKERS = NUM_CORES * NUM_SUBCORES
CHUNK = 128
CPG = 1
NG = 5
GROUP = CPG * CHUNK


def _make_gather(total, vocab):
    assert total % (NUM_WORKERS * GROUP) == 0
    per_worker = total // NUM_WORKERS
    n_chunks = per_worker // CHUNK
    n_groups = n_chunks // CPG
    n_steady = (n_groups // NG) - 1
    assert n_steady >= 1

    mesh = plsc.VectorSubcoreMesh(
        core_axis_name="c",
        subcore_axis_name="s",
        num_cores=NUM_CORES,
        num_subcores=NUM_SUBCORES,
    )

    @functools.partial(
        pl.kernel,
        out_type=jax.ShapeDtypeStruct((total, DIM), jnp.float32),
        mesh=mesh,
        scratch_types=[
            pltpu.VMEM((n_chunks, CHUNK), jnp.int32),
            pltpu.VMEM((NG * GROUP, DIM), jnp.float32),
            [pltpu.SemaphoreType.DMA for _ in range(NG)],
            [pltpu.SemaphoreType.DMA for _ in range(NG)],
        ],
    )
    def gather(idx_hbm, table_hbm, out_hbm, idx_v, rows, gsem, osem):
        wid = lax.axis_index("s") * NUM_CORES + lax.axis_index("c")
        base = wid * per_worker

        pltpu.sync_copy(idx_hbm.at[pl.ds(wid * n_chunks, n_chunks)], idx_v)

        def gathers(p, b):
            return [
                pltpu.make_async_copy(
                    table_hbm.at[idx_v.at[p * CPG + j]],
                    rows.at[pl.ds(b * GROUP + j * CHUNK, CHUNK)],
                    gsem[b],
                )
                for j in range(CPG)
            ]

        def out_copy(p, b):
            return pltpu.make_async_copy(
                rows.at[pl.ds(b * GROUP, GROUP)],
                out_hbm.at[pl.ds(base + p * GROUP, GROUP)],
                osem[b],
            )

        def step(p, b, first_round):
            if not first_round:
                out_copy(p - NG, b).wait()
            for c in gathers(p, b):
                c.start()
            if not (first_round and b == 0):
                u, bu = p - 1, (b - 1) % NG
                for c in gathers(u, bu):
                    c.wait()
                out_copy(u, bu).start()

        for b in range(NG):
            step(b, b, first_round=True)

        def outer(g):
            for b in range(NG):
                step(g * NG + b, b, first_round=False)

        pl.loop(1, n_steady + 1)(outer)

        for p in range((n_steady + 1) * NG, n_groups):
            step(p, p % NG, first_round=False)
        u, bu = n_groups - 1, (n_groups - 1) % NG
        for c in gathers(u, bu):
            c.wait()
        out_copy(u, bu).start()
        for k in range(NG):
            p = n_groups - NG + k
            out_copy(p, p % NG).wait()

    return gather


def kernel(inputs, embeddings):
    batch, hist = inputs.shape
    idx = inputs.reshape(-1).astype(jnp.int32)
    total = idx.shape[0]
    idx2d = idx.reshape(total // CHUNK, CHUNK)
    out = _make_gather(total, embeddings.shape[0])(idx2d, embeddings)
    return out.reshape(batch, hist, DIM)


# speedup vs baseline: 1.2400x; 1.0017x over previous
import functools

import jax
import jax.numpy as jnp
from jax import lax
from jax.experimental import pallas as pl
from jax.experimental.pallas import tpu as pltpu
from jax.experimental.pallas import tpu_sc as plsc

DIM = 128
NUM_CORES = 2
NUM_SUBCORES = 16
NUM_WORKERS = NUM_CORES * NUM_SUBCORES
CHUNK = 128
CPG = 1
NG = 6
GLAG = 3
GROUP = CPG * CHUNK


def _make_gather(total, vocab):
    assert total % (NUM_WORKERS * GROUP) == 0
    per_worker = total // NUM_WORKERS
    n_chunks = per_worker // CHUNK
    n_groups = n_chunks // CPG
    n_steady = (n_groups // NG) - 1
    assert n_steady >= 1

    mesh = plsc.VectorSubcoreMesh(
        core_axis_name="c",
        subcore_axis_name="s",
        num_cores=NUM_CORES,
        num_subcores=NUM_SUBCORES,
    )

    @functools.partial(
        pl.kernel,
        out_type=jax.ShapeDtypeStruct((total, DIM), jnp.float32),
        mesh=mesh,
        scratch_types=[
            pltpu.VMEM((n_chunks, CHUNK), jnp.int32),
            pltpu.VMEM((NG * GROUP, DIM), jnp.float32),
            [pltpu.SemaphoreType.DMA for _ in range(NG)],
            [pltpu.SemaphoreType.DMA for _ in range(NG)],
        ],
    )
    def gather(idx_hbm, table_hbm, out_hbm, idx_v, rows, gsem, osem):
        wid = lax.axis_index("s") * NUM_CORES + lax.axis_index("c")
        base = wid * per_worker

        pltpu.sync_copy(idx_hbm.at[pl.ds(wid * n_chunks, n_chunks)], idx_v)

        def gathers(p, b):
            return [
                pltpu.make_async_copy(
                    table_hbm.at[idx_v.at[p * CPG + j]],
                    rows.at[pl.ds(b * GROUP + j * CHUNK, CHUNK)],
                    gsem[b],
                )
                for j in range(CPG)
            ]

        def out_copy(p, b):
            return pltpu.make_async_copy(
                rows.at[pl.ds(b * GROUP, GROUP)],
                out_hbm.at[pl.ds(base + p * GROUP, GROUP)],
                osem[b],
            )

        def step(p, b, first_round):
            if not first_round:
                out_copy(p - NG, b).wait()
            for c in gathers(p, b):
                c.start()
            if not (first_round and b < GLAG):
                u, bu = p - GLAG, (b - GLAG) % NG
                for c in gathers(u, bu):
                    c.wait()
                out_copy(u, bu).start()

        for b in range(NG):
            step(b, b, first_round=True)

        def outer(g):
            for b in range(NG):
                step(g * NG + b, b, first_round=False)

        pl.loop(1, n_steady + 1)(outer)

        for p in range((n_steady + 1) * NG, n_groups):
            step(p, p % NG, first_round=False)
        for u in range(n_groups - GLAG, n_groups):
            bu = u % NG
            for c in gathers(u, bu):
                c.wait()
            out_copy(u, bu).start()
        for k in range(NG):
            p = n_groups - NG + k
            out_copy(p, p % NG).wait()

    return gather


def kernel(inputs, embeddings):
    batch, hist = inputs.shape
    idx = inputs.reshape(-1).astype(jnp.int32)
    total = idx.shape[0]
    idx2d = idx.reshape(total // CHUNK, CHUNK)
    out = _make_gather(total, embeddings.shape[0])(idx2d, embeddings)
    return out.reshape(batch, hist, DIM)
